# baseline (speedup 1.0000x reference)
import functools

import jax
import jax.numpy as jnp
from jax import lax
from jax.experimental import pallas as pl
from jax.experimental.pallas import tpu as pltpu
from jax.experimental.pallas import tpu_sc as plsc

NC, NS = 2, 16
NW = NC * NS
CHUNK = 128


def _sc_mesh():
    return plsc.VectorSubcoreMesh(core_axis_name="c", subcore_axis_name="s")


NBUF = 8
KLAG = 6


def _make_deg(npad, nchunk):
    @functools.partial(
        pl.kernel,
        out_type=jax.ShapeDtypeStruct((NC, npad, 8), jnp.float32),
        mesh=_sc_mesh(),
        scratch_types=[
            pltpu.VMEM((nchunk, CHUNK), jnp.int32),
            pltpu.VMEM((CHUNK, 8), jnp.float32),
            pltpu.VMEM((CHUNK, 8), jnp.float32),
            pltpu.VMEM_SHARED((npad, 8), jnp.float32),
        ] + [pltpu.SemaphoreType.DMA] * NBUF,
        compiler_params=pltpu.CompilerParams(use_tc_tiling_on_sc=False),
    )
    def deg_kernel(dst_hbm, ones_hbm, zeros_hbm, out_hbm, idxb, onesb, zbuf,
                   acc_sh, *ssems):
        c = lax.axis_index("c")
        s = lax.axis_index("s")

        w = c * NS + s
        pltpu.sync_copy(dst_hbm.at[w], idxb)
        pltpu.sync_copy(ones_hbm, onesb)
        pltpu.sync_copy(zeros_hbm, zbuf)
        rpt0 = npad // NS
        for q in range(rpt0 // CHUNK):
            pltpu.sync_copy(zbuf, acc_sh.at[pl.ds(s * rpt0 + q * CHUNK,
                                                  CHUNK)])
        plsc.subcore_barrier()

        def outer(j, carry):
            for b in range(NBUF):
                i = j * NBUF + b
                desc = pltpu.async_copy(onesb, acc_sh.at[idxb.at[i]],
                                        ssems[b], add=True)

                @pl.when(j > 0)
                def _():
                    pltpu.make_async_copy(onesb, acc_sh.at[idxb.at[i]],
                                          ssems[b]).wait()
                del desc
            return carry

        lax.fori_loop(0, nchunk // NBUF, outer, 0)
        for b in range(NBUF):
            pltpu.make_async_copy(onesb, acc_sh.at[idxb.at[b]], ssems[b]).wait()
        plsc.subcore_barrier()
        rpt = npad // NS
        pltpu.sync_copy(acc_sh.at[pl.ds(s * rpt, rpt)],
                        out_hbm.at[c, pl.ds(s * rpt, rpt)])

    return deg_kernel


def _make_agg(npad, nchunk, width, nrows, nh):
    @functools.partial(
        pl.kernel,
        out_type=jax.ShapeDtypeStruct((NC, nh, npad, width), jnp.float32),
        mesh=_sc_mesh(),
        scratch_types=[
            pltpu.VMEM((nchunk, CHUNK), jnp.int32),
            pltpu.VMEM((nchunk, CHUNK), jnp.int32),
        ] + [pltpu.VMEM((CHUNK, width), jnp.float32)] * (NBUF + 1)
          + [pltpu.SemaphoreType.DMA] * (2 * NBUF)
          + [pltpu.VMEM_SHARED((npad, width), jnp.float32)]
          + [pltpu.VMEM_SHARED((nrows, width), jnp.float32)] * nh,
        compiler_params=pltpu.CompilerParams(use_tc_tiling_on_sc=False),
    )
    def agg_kernel(src_hbm, dst_hbm, *rest):
        hs_hbm = rest[:nh]
        zeros_hbm, out_hbm, srcb, dstb = rest[nh:nh + 4]
        rows_and_sems = rest[nh + 4:]
        rows = rows_and_sems[:NBUF]
        zbuf = rows_and_sems[NBUF]
        gsems = rows_and_sems[NBUF + 1:2 * NBUF + 1]
        ssems = rows_and_sems[2 * NBUF + 1:3 * NBUF + 1]
        acc_sh = rows_and_sems[3 * NBUF + 1]
        hs_sh = rows_and_sems[3 * NBUF + 2:]
        c = lax.axis_index("c")
        s = lax.axis_index("s")

        piece = nrows // 10
        for p in range(10 * nh):
            @pl.when(s == p % NS)
            def _():
                hh, r0 = p // 10, (p % 10) * piece
                pltpu.sync_copy(hs_hbm[hh].at[pl.ds(r0, piece)],
                                hs_sh[hh].at[pl.ds(r0, piece)])

        w = c * NS + s
        pltpu.sync_copy(src_hbm.at[w], srcb)
        pltpu.sync_copy(dst_hbm.at[w], dstb)

        pltpu.sync_copy(zeros_hbm, zbuf)

        rpt = npad // NS
        for hh in range(nh):
            h_sh = hs_sh[hh]
            plsc.subcore_barrier()
            for q in range(rpt // CHUNK):
                pltpu.sync_copy(zbuf, acc_sh.at[pl.ds(s * rpt + q * CHUNK,
                                                      CHUNK)])
            plsc.subcore_barrier()

            for b in range(NBUF):
                pltpu.async_copy(h_sh.at[srcb.at[b]], rows[b], gsems[b])

            def outer(jj, carry):
                for b in range(NBUF):
                    i = jj * NBUF + b
                    pltpu.make_async_copy(h_sh.at[srcb.at[i]], rows[b],
                                          gsems[b]).wait()
                    pltpu.async_copy(rows[b], acc_sh.at[dstb.at[i]],
                                     ssems[b], add=True)
                    jbuf = (b - KLAG) % NBUF
                    jc = i - KLAG

                    @pl.when(jc >= 0)
                    def _():
                        pltpu.make_async_copy(rows[jbuf],
                                              acc_sh.at[dstb.at[0]],
                                              ssems[jbuf]).wait()

                    @pl.when((jc >= 0) & (jc + NBUF < nchunk))
                    def _():
                        pltpu.async_copy(h_sh.at[srcb.at[jc + NBUF]],
                                         rows[jbuf], gsems[jbuf])
                return carry

            lax.fori_loop(0, nchunk // NBUF, outer, 0)
            for t in range(KLAG):
                b = (nchunk - KLAG + t) % NBUF
                pltpu.make_async_copy(rows[b], acc_sh.at[dstb.at[0]],
                                      ssems[b]).wait()
            plsc.subcore_barrier()
            pltpu.sync_copy(acc_sh.at[pl.ds(s * rpt, rpt)],
                            out_hbm.at[c, hh, pl.ds(s * rpt, rpt)])

    return agg_kernel


def _elu(z):
    return jnp.where(z > 0, z, jnp.exp(jnp.minimum(z, 0.0)) - 1.0)


def _tc_mm(x, W1, bm=5000):
    n, fin = x.shape
    h = W1.shape[1]

    def body(x_ref, w_ref, h_ref):
        h_ref[...] = jnp.dot(x_ref[...], w_ref[...],
                             preferred_element_type=jnp.float32)

    return pl.pallas_call(
        body,
        grid=(n // bm,),
        in_specs=[
            pl.BlockSpec((bm, fin), lambda i: (i, 0)),
            pl.BlockSpec((fin, h), lambda i: (0, 0)),
        ],
        out_specs=pl.BlockSpec((bm, h), lambda i: (i, 0)),
        out_shape=jax.ShapeDtypeStruct((n, h), jnp.float32),
    )(x, W1)


def _tc_scale(deg_parts, h1, bm=5000):
    n, h = h1.shape

    def body(deg_ref, h_ref, dinv_ref, lo_ref, hi_ref):
        degsum = deg_ref[0, :, 0:1] + deg_ref[1, :, 0:1] + 1.0
        dinv = lax.rsqrt(degsum)
        dinv_ref[...] = dinv
        hp = h_ref[...] * dinv
        lo_ref[...] = hp[:, :h // 2]
        hi_ref[...] = hp[:, h // 2:]

    return pl.pallas_call(
        body,
        grid=(n // bm,),
        in_specs=[
            pl.BlockSpec((NC, bm, 8), lambda i: (0, i, 0)),
            pl.BlockSpec((bm, h), lambda i: (i, 0)),
        ],
        out_specs=[
            pl.BlockSpec((bm, 1), lambda i: (i, 0)),
            pl.BlockSpec((bm, h // 2), lambda i: (i, 0)),
            pl.BlockSpec((bm, h // 2), lambda i: (i, 0)),
        ],
        out_shape=[
            jax.ShapeDtypeStruct((n, 1), jnp.float32),
            jax.ShapeDtypeStruct((n, h // 2), jnp.float32),
            jax.ShapeDtypeStruct((n, h // 2), jnp.float32),
        ],
    )(deg_parts, h1)


def _tc2(parts1, hp_lo, hp_hi, dinv, b1r, W2p, bm=5000):
    n, hh = hp_lo.shape
    h = 2 * hh
    wout = W2p.shape[1]

    def body(p_ref, lo_ref, hi_ref, dinv_ref, b1_ref, w2_ref, h2p_ref):
        lo = p_ref[0, 0] + p_ref[1, 0] + lo_ref[...]
        hi = p_ref[0, 1] + p_ref[1, 1] + hi_ref[...]
        agg = jnp.concatenate([lo, hi], axis=-1)
        z = dinv_ref[...] * agg + b1_ref[...]
        out1 = _elu(z)
        h2p_ref[...] = jnp.dot(out1, w2_ref[...],
                               preferred_element_type=jnp.float32) * dinv_ref[...]

    return pl.pallas_call(
        body,
        grid=(n // bm,),
        in_specs=[
            pl.BlockSpec((NC, 2, bm, hh), lambda i: (0, 0, i, 0)),
            pl.BlockSpec((bm, hh), lambda i: (i, 0)),
            pl.BlockSpec((bm, hh), lambda i: (i, 0)),
            pl.BlockSpec((bm, 1), lambda i: (i, 0)),
            pl.BlockSpec((1, h), lambda i: (0, 0)),
            pl.BlockSpec((h, wout), lambda i: (0, 0)),
        ],
        out_specs=pl.BlockSpec((bm, wout), lambda i: (i, 0)),
        out_shape=jax.ShapeDtypeStruct((n, wout), jnp.float32),
    )(parts1, hp_lo, hp_hi, dinv, b1r, W2p)


def _tc3(parts2, h2p, dinv, b2r, bm=5000):
    n, w = h2p.shape

    def body(p_ref, h2p_ref, dinv_ref, b2_ref, out_ref):
        agg = p_ref[0, 0] + p_ref[1, 0] + h2p_ref[...]
        z = dinv_ref[...] * agg + b2_ref[...]
        out_ref[...] = _elu(z)

    return pl.pallas_call(
        body,
        grid=(n // bm,),
        in_specs=[
            pl.BlockSpec((NC, 1, bm, w), lambda i: (0, 0, i, 0)),
            pl.BlockSpec((bm, w), lambda i: (i, 0)),
            pl.BlockSpec((bm, 1), lambda i: (i, 0)),
            pl.BlockSpec((1, w), lambda i: (0, 0)),
        ],
        out_specs=pl.BlockSpec((bm, w), lambda i: (i, 0)),
        out_shape=jax.ShapeDtypeStruct((n, w), jnp.float32),
    )(parts2, h2p, dinv, b2r)


def kernel(x, adj, num_graphs, in_batch, cluster, W1, b1, W2, b2):
    n, fin = x.shape
    hdim = W1.shape[1]
    fout = W2.shape[1]
    e = adj.shape[1]

    ept = -(-e // (NW * CHUNK * NBUF)) * (CHUNK * NBUF)
    ep = ept * NW
    nchunk = ept // CHUNK
    npad = -(-(n + 1) // (NS * 8)) * (NS * 8) + NS * 8

    src = jnp.concatenate([adj[0], jnp.zeros((ep - e,), jnp.int32)])
    dump = n + jnp.arange(ep - e, dtype=jnp.int32) % (npad - n)
    dst = jnp.concatenate([adj[1], dump])
    src3 = src.reshape(NW, nchunk, CHUNK)
    dst3 = dst.reshape(NW, nchunk, CHUNK)

    ones8 = jnp.ones((CHUNK, 8), jnp.float32)
    zeros8 = jnp.zeros((CHUNK, 8), jnp.float32)
    zerosW = jnp.zeros((CHUNK, hdim // 2), jnp.float32)

    deg_parts = _make_deg(npad, nchunk)(dst3, ones8, zeros8)
    h1 = _tc_mm(x, W1)
    dinv, hp_lo, hp_hi = _tc_scale(deg_parts, h1)
    parts1 = _make_agg(npad, nchunk, hdim // 2, n, 2)(src3, dst3, hp_lo, hp_hi,
                                                      zerosW)
    W2p = jnp.pad(W2, ((0, 0), (0, 8 - fout)))
    b2p = jnp.pad(b2, (0, 8 - fout)).reshape(1, 8)
    h2p = _tc2(parts1, hp_lo, hp_hi, dinv, b1.reshape(1, hdim), W2p)
    parts2 = _make_agg(npad, nchunk, 8, n, 1)(src3, dst3, h2p, zeros8)
    out8 = _tc3(parts2, h2p, dinv, b2p)
    return out8[:, :fout]


# speedup vs baseline: 1.0121x; 1.0121x over previous
import functools

import jax
import jax.numpy as jnp
from jax import lax
from jax.experimental import pallas as pl
from jax.experimental.pallas import tpu as pltpu
from jax.experimental.pallas import tpu_sc as plsc

NC, NS = 2, 16
NW = NC * NS
CHUNK = 128


def _sc_mesh():
    return plsc.VectorSubcoreMesh(core_axis_name="c", subcore_axis_name="s")


NBUF = 8
KLAG = 4


def _make_deg(npad, nchunk):
    @functools.partial(
        pl.kernel,
        out_type=jax.ShapeDtypeStruct((NC, npad, 8), jnp.float32),
        mesh=_sc_mesh(),
        scratch_types=[
            pltpu.VMEM((nchunk, CHUNK), jnp.int32),
            pltpu.VMEM((CHUNK, 8), jnp.float32),
            pltpu.VMEM((CHUNK, 8), jnp.float32),
            pltpu.VMEM_SHARED((npad, 8), jnp.float32),
        ] + [pltpu.SemaphoreType.DMA] * NBUF,
        compiler_params=pltpu.CompilerParams(use_tc_tiling_on_sc=False),
    )
    def deg_kernel(dst_hbm, ones_hbm, zeros_hbm, out_hbm, idxb, onesb, zbuf,
                   acc_sh, *ssems):
        c = lax.axis_index("c")
        s = lax.axis_index("s")

        w = c * NS + s
        pltpu.sync_copy(dst_hbm.at[w], idxb)
        pltpu.sync_copy(ones_hbm, onesb)
        pltpu.sync_copy(zeros_hbm, zbuf)
        rpt0 = npad // NS
        for q in range(rpt0 // CHUNK):
            pltpu.sync_copy(zbuf, acc_sh.at[pl.ds(s * rpt0 + q * CHUNK,
                                                  CHUNK)])
        plsc.subcore_barrier()

        def outer(j, carry):
            for b in range(NBUF):
                i = j * NBUF + b
                desc = pltpu.async_copy(onesb, acc_sh.at[idxb.at[i]],
                                        ssems[b], add=True)

                @pl.when(j > 0)
                def _():
                    pltpu.make_async_copy(onesb, acc_sh.at[idxb.at[i]],
                                          ssems[b]).wait()
                del desc
            return carry

        lax.fori_loop(0, nchunk // NBUF, outer, 0)
        for b in range(NBUF):
            pltpu.make_async_copy(onesb, acc_sh.at[idxb.at[b]], ssems[b]).wait()
        plsc.subcore_barrier()
        rpt = npad // NS
        pltpu.sync_copy(acc_sh.at[pl.ds(s * rpt, rpt)],
                        out_hbm.at[c, pl.ds(s * rpt, rpt)])

    return deg_kernel


def _make_agg(npad, nchunk, width, nrows, nh):
    @functools.partial(
        pl.kernel,
        out_type=jax.ShapeDtypeStruct((NC, nh, npad, width), jnp.float32),
        mesh=_sc_mesh(),
        scratch_types=[
            pltpu.VMEM((nchunk, CHUNK), jnp.int32),
            pltpu.VMEM((nchunk, CHUNK), jnp.int32),
        ] + [pltpu.VMEM((CHUNK, width), jnp.float32)] * (NBUF + 1)
          + [pltpu.SemaphoreType.DMA] * (2 * NBUF)
          + [pltpu.VMEM_SHARED((npad, width), jnp.float32)]
          + [pltpu.VMEM_SHARED((nrows, width), jnp.float32)] * nh,
        compiler_params=pltpu.CompilerParams(use_tc_tiling_on_sc=False),
    )
    def agg_kernel(src_hbm, dst_hbm, *rest):
        hs_hbm = rest[:nh]
        zeros_hbm, out_hbm, srcb, dstb = rest[nh:nh + 4]
        rows_and_sems = rest[nh + 4:]
        rows = rows_and_sems[:NBUF]
        zbuf = rows_and_sems[NBUF]
        gsems = rows_and_sems[NBUF + 1:2 * NBUF + 1]
        ssems = rows_and_sems[2 * NBUF + 1:3 * NBUF + 1]
        acc_sh = rows_and_sems[3 * NBUF + 1]
        hs_sh = rows_and_sems[3 * NBUF + 2:]
        c = lax.axis_index("c")
        s = lax.axis_index("s")

        piece = nrows // 10
        for p in range(10 * nh):
            @pl.when(s == p % NS)
            def _():
                hh, r0 = p // 10, (p % 10) * piece
                pltpu.sync_copy(hs_hbm[hh].at[pl.ds(r0, piece)],
                                hs_sh[hh].at[pl.ds(r0, piece)])

        w = c * NS + s
        pltpu.sync_copy(src_hbm.at[w], srcb)
        pltpu.sync_copy(dst_hbm.at[w], dstb)

        pltpu.sync_copy(zeros_hbm, zbuf)

        rpt = npad // NS
        for hh in range(nh):
            h_sh = hs_sh[hh]
            plsc.subcore_barrier()
            for q in range(rpt // CHUNK):
                pltpu.sync_copy(zbuf, acc_sh.at[pl.ds(s * rpt + q * CHUNK,
                                                      CHUNK)])
            plsc.subcore_barrier()

            for b in range(NBUF):
                pltpu.async_copy(h_sh.at[srcb.at[b]], rows[b], gsems[b])

            def outer(jj, carry):
                for b in range(NBUF):
                    i = jj * NBUF + b
                    pltpu.make_async_copy(h_sh.at[srcb.at[i]], rows[b],
                                          gsems[b]).wait()
                    pltpu.async_copy(rows[b], acc_sh.at[dstb.at[i]],
                                     ssems[b], add=True)
                    jbuf = (b - KLAG) % NBUF
                    jc = i - KLAG

                    @pl.when(jc >= 0)
                    def _():
                        pltpu.make_async_copy(rows[jbuf],
                                              acc_sh.at[dstb.at[0]],
                                              ssems[jbuf]).wait()

                    @pl.when((jc >= 0) & (jc + NBUF < nchunk))
                    def _():
                        pltpu.async_copy(h_sh.at[srcb.at[jc + NBUF]],
                                         rows[jbuf], gsems[jbuf])
                return carry

            lax.fori_loop(0, nchunk // NBUF, outer, 0)
            for t in range(KLAG):
                b = (nchunk - KLAG + t) % NBUF
                pltpu.make_async_copy(rows[b], acc_sh.at[dstb.at[0]],
                                      ssems[b]).wait()
            plsc.subcore_barrier()
            pltpu.sync_copy(acc_sh.at[pl.ds(s * rpt, rpt)],
                            out_hbm.at[c, hh, pl.ds(s * rpt, rpt)])

    return agg_kernel


def _elu(z):
    return jnp.where(z > 0, z, jnp.exp(jnp.minimum(z, 0.0)) - 1.0)


def _tc_mm(x, W1, bm=5000):
    n, fin = x.shape
    h = W1.shape[1]

    def body(x_ref, w_ref, h_ref):
        h_ref[...] = jnp.dot(x_ref[...], w_ref[...],
                             preferred_element_type=jnp.float32)

    return pl.pallas_call(
        body,
        grid=(n // bm,),
        in_specs=[
            pl.BlockSpec((bm, fin), lambda i: (i, 0)),
            pl.BlockSpec((fin, h), lambda i: (0, 0)),
        ],
        out_specs=pl.BlockSpec((bm, h), lambda i: (i, 0)),
        out_shape=jax.ShapeDtypeStruct((n, h), jnp.float32),
    )(x, W1)


def _tc_scale(deg_parts, h1, bm=5000):
    n, h = h1.shape

    def body(deg_ref, h_ref, dinv_ref, lo_ref, hi_ref):
        degsum = deg_ref[0, :, 0:1] + deg_ref[1, :, 0:1] + 1.0
        dinv = lax.rsqrt(degsum)
        dinv_ref[...] = dinv
        hp = h_ref[...] * dinv
        lo_ref[...] = hp[:, :h // 2]
        hi_ref[...] = hp[:, h // 2:]

    return pl.pallas_call(
        body,
        grid=(n // bm,),
        in_specs=[
            pl.BlockSpec((NC, bm, 8), lambda i: (0, i, 0)),
            pl.BlockSpec((bm, h), lambda i: (i, 0)),
        ],
        out_specs=[
            pl.BlockSpec((bm, 1), lambda i: (i, 0)),
            pl.BlockSpec((bm, h // 2), lambda i: (i, 0)),
            pl.BlockSpec((bm, h // 2), lambda i: (i, 0)),
        ],
        out_shape=[
            jax.ShapeDtypeStruct((n, 1), jnp.float32),
            jax.ShapeDtypeStruct((n, h // 2), jnp.float32),
            jax.ShapeDtypeStruct((n, h // 2), jnp.float32),
        ],
    )(deg_parts, h1)


def _tc2(parts1, hp_lo, hp_hi, dinv, b1r, W2p, bm=5000):
    n, hh = hp_lo.shape
    h = 2 * hh
    wout = W2p.shape[1]

    def body(p_ref, lo_ref, hi_ref, dinv_ref, b1_ref, w2_ref, h2p_ref):
        lo = p_ref[0, 0] + p_ref[1, 0] + lo_ref[...]
        hi = p_ref[0, 1] + p_ref[1, 1] + hi_ref[...]
        agg = jnp.concatenate([lo, hi], axis=-1)
        z = dinv_ref[...] * agg + b1_ref[...]
        out1 = _elu(z)
        h2p_ref[...] = jnp.dot(out1, w2_ref[...],
                               preferred_element_type=jnp.float32) * dinv_ref[...]

    return pl.pallas_call(
        body,
        grid=(n // bm,),
        in_specs=[
            pl.BlockSpec((NC, 2, bm, hh), lambda i: (0, 0, i, 0)),
            pl.BlockSpec((bm, hh), lambda i: (i, 0)),
            pl.BlockSpec((bm, hh), lambda i: (i, 0)),
            pl.BlockSpec((bm, 1), lambda i: (i, 0)),
            pl.BlockSpec((1, h), lambda i: (0, 0)),
            pl.BlockSpec((h, wout), lambda i: (0, 0)),
        ],
        out_specs=pl.BlockSpec((bm, wout), lambda i: (i, 0)),
        out_shape=jax.ShapeDtypeStruct((n, wout), jnp.float32),
    )(parts1, hp_lo, hp_hi, dinv, b1r, W2p)


def _tc3(parts2, h2p, dinv, b2r, bm=5000):
    n, w = h2p.shape

    def body(p_ref, h2p_ref, dinv_ref, b2_ref, out_ref):
        agg = p_ref[0, 0] + p_ref[1, 0] + h2p_ref[...]
        z = dinv_ref[...] * agg + b2_ref[...]
        out_ref[...] = _elu(z)

    return pl.pallas_call(
        body,
        grid=(n // bm,),
        in_specs=[
            pl.BlockSpec((NC, 1, bm, w), lambda i: (0, 0, i, 0)),
            pl.BlockSpec((bm, w), lambda i: (i, 0)),
            pl.BlockSpec((bm, 1), lambda i: (i, 0)),
            pl.BlockSpec((1, w), lambda i: (0, 0)),
        ],
        out_specs=pl.BlockSpec((bm, w), lambda i: (i, 0)),
        out_shape=jax.ShapeDtypeStruct((n, w), jnp.float32),
    )(parts2, h2p, dinv, b2r)


def kernel(x, adj, num_graphs, in_batch, cluster, W1, b1, W2, b2):
    n, fin = x.shape
    hdim = W1.shape[1]
    fout = W2.shape[1]
    e = adj.shape[1]

    ept = -(-e // (NW * CHUNK * NBUF)) * (CHUNK * NBUF)
    ep = ept * NW
    nchunk = ept // CHUNK
    npad = -(-(n + 1) // (NS * 8)) * (NS * 8) + NS * 8

    src = jnp.concatenate([adj[0], jnp.zeros((ep - e,), jnp.int32)])
    dump = n + jnp.arange(ep - e, dtype=jnp.int32) % (npad - n)
    dst = jnp.concatenate([adj[1], dump])
    src3 = src.reshape(NW, nchunk, CHUNK)
    dst3 = dst.reshape(NW, nchunk, CHUNK)

    ones8 = jnp.ones((CHUNK, 8), jnp.float32)
    zeros8 = jnp.zeros((CHUNK, 8), jnp.float32)
    zerosW = jnp.zeros((CHUNK, hdim // 2), jnp.float32)

    deg_parts = _make_deg(npad, nchunk)(dst3, ones8, zeros8)
    h1 = _tc_mm(x, W1)
    dinv, hp_lo, hp_hi = _tc_scale(deg_parts, h1)
    parts1 = _make_agg(npad, nchunk, hdim // 2, n, 2)(src3, dst3, hp_lo, hp_hi,
                                                      zerosW)
    W2p = jnp.pad(W2, ((0, 0), (0, 8 - fout)))
    b2p = jnp.pad(b2, (0, 8 - fout)).reshape(1, 8)
    h2p = _tc2(parts1, hp_lo, hp_hi, dinv, b1.reshape(1, hdim), W2p)
    parts2 = _make_agg(npad, nchunk, 8, n, 1)(src3, dst3, h2p, zeros8)
    out8 = _tc3(parts2, h2p, dinv, b2p)
    return out8[:, :fout]
